# baseline (speedup 1.0000x reference)
import functools

import jax
import jax.numpy as jnp
from jax import lax
from jax.experimental import pallas as pl
from jax.experimental.pallas import tpu as pltpu
from jax.experimental.pallas import tpu_sc as plsc

N_NODES = 10000
D_IN = 256
D_HID = 128
D_OUT = 256

NC = 2
NS = 16
CHUNK = 128

N_PAD = 10240
E_PAD = 163840
DUMMY = N_NODES

ROW_BLK = 1024
OUT_BLK = 1000
DEG_W = 128
N_CHUNKS = E_PAD // (NC * NS) // CHUNK


def _vector_mesh():
    return plsc.VectorSubcoreMesh(core_axis_name="c", subcore_axis_name="s")


def _sc_degree(ei3, ones_hbm, zeros_hbm):
    @functools.partial(
        pl.kernel,
        mesh=_vector_mesh(),
        out_type=jax.ShapeDtypeStruct((NC, N_PAD, DEG_W), jnp.float32),
        scratch_types=[
            pltpu.VMEM((N_CHUNKS, CHUNK), jnp.int32),
            pltpu.VMEM((CHUNK, DEG_W), jnp.float32),
            pltpu.VMEM_SHARED((N_PAD, DEG_W), jnp.float32),
            pltpu.SemaphoreType.DMA,
        ],
    )
    def k(ei_h, ones_h, zeros_h, out_h, dst_v, ones_v, acc_sh, sem):
        cid = lax.axis_index("c")
        sid = lax.axis_index("s")
        wid = cid * NS + sid
        rpt = N_PAD // NS
        pltpu.sync_copy(zeros_h.at[pl.ds(sid * rpt, rpt)],
                        acc_sh.at[pl.ds(sid * rpt, rpt)])
        pltpu.sync_copy(ones_h, ones_v)
        pltpu.sync_copy(ei_h.at[1, wid], dst_v)
        plsc.subcore_barrier()

        @pl.loop(0, N_CHUNKS)
        def _(j):
            pltpu.async_copy(ones_v, acc_sh.at[dst_v.at[j]], sem, add=True)

        @pl.loop(0, N_CHUNKS)
        def _(j):
            pltpu.make_async_copy(ones_v, acc_sh.at[dst_v.at[j]], sem).wait()

        plsc.subcore_barrier()
        pltpu.sync_copy(acc_sh.at[pl.ds(sid * rpt, rpt)],
                        out_h.at[cid, pl.ds(sid * rpt, rpt)])

    return k(ei3, ones_hbm, zeros_hbm)


def _sc_edge_pass(table, ei3, zeros_hbm):
    NBUF = 2

    @functools.partial(
        pl.kernel,
        mesh=_vector_mesh(),
        out_type=jax.ShapeDtypeStruct((NC, N_PAD, D_HID), jnp.float32),
        scratch_types=[
            pltpu.VMEM((N_CHUNKS, CHUNK), jnp.int32),
            pltpu.VMEM((N_CHUNKS, CHUNK), jnp.int32),
            pltpu.VMEM((NBUF, CHUNK, D_HID), jnp.float32),
            pltpu.VMEM_SHARED((N_PAD, D_HID), jnp.float32),
        ]
        + [pltpu.SemaphoreType.DMA] * (2 * NBUF),
    )
    def k(tab_h, ei_h, zeros_h, out_h, src_v, dst_v, rows, acc_sh, *sems):
        semg = sems[:NBUF]
        semw = sems[NBUF:]
        cid = lax.axis_index("c")
        sid = lax.axis_index("s")
        wid = cid * NS + sid
        rpt = N_PAD // NS
        pltpu.sync_copy(zeros_h.at[pl.ds(sid * rpt, rpt)],
                        acc_sh.at[pl.ds(sid * rpt, rpt)])
        pltpu.sync_copy(ei_h.at[0, wid], src_v)
        pltpu.sync_copy(ei_h.at[1, wid], dst_v)
        plsc.subcore_barrier()

        for b in range(NBUF):
            pltpu.async_copy(tab_h.at[src_v.at[b]], rows.at[b], semg[b])

        @pl.loop(0, N_CHUNKS - NBUF, step=NBUF)
        def _(j):
            for b in range(NBUF):
                pltpu.make_async_copy(tab_h.at[src_v.at[j + b]], rows.at[b],
                                      semg[b]).wait()
                pltpu.async_copy(rows.at[b], acc_sh.at[dst_v.at[j + b]],
                                 semw[b], add=True)
            for b in range(NBUF):
                pltpu.make_async_copy(rows.at[b],
                                      acc_sh.at[dst_v.at[j + b]],
                                      semw[b]).wait()
                pltpu.async_copy(tab_h.at[src_v.at[j + NBUF + b]],
                                 rows.at[b], semg[b])

        j0 = N_CHUNKS - NBUF
        for b in range(NBUF):
            pltpu.make_async_copy(tab_h.at[src_v.at[j0 + b]], rows.at[b],
                                  semg[b]).wait()
            pltpu.async_copy(rows.at[b], acc_sh.at[dst_v.at[j0 + b]],
                             semw[b], add=True)
        for b in range(NBUF):
            pltpu.make_async_copy(rows.at[b], acc_sh.at[dst_v.at[j0 + b]],
                                  semw[b]).wait()

        plsc.subcore_barrier()
        pltpu.sync_copy(acc_sh.at[pl.ds(sid * rpt, rpt)],
                        out_h.at[cid, pl.ds(sid * rpt, rpt)])

    return k(table, ei3, zeros_hbm)


def _tc_mm0(x_pad, W0):
    def body(x_ref, w_ref, h_ref):
        h_ref[...] = jnp.dot(x_ref[...], w_ref[...],
                             preferred_element_type=jnp.float32)

    return pl.pallas_call(
        body,
        grid=(N_PAD // ROW_BLK,),
        in_specs=[
            pl.BlockSpec((ROW_BLK, D_IN), lambda i: (i, 0)),
            pl.BlockSpec((D_IN, D_HID), lambda i: (0, 0)),
        ],
        out_specs=pl.BlockSpec((ROW_BLK, D_HID), lambda i: (i, 0)),
        out_shape=jax.ShapeDtypeStruct((N_PAD, D_HID), jnp.float32),
    )(x_pad, W0)


def _tc_scale(h0, deg_parts):
    def body(h0_ref, dg_ref, h_ref, dis_ref):
        cnt = dg_ref[0, :, 0:1] + dg_ref[1, :, 0:1]
        dis = lax.rsqrt(cnt + 1.0)
        h_ref[...] = h0_ref[...] * dis
        dis_ref[...] = dis

    return pl.pallas_call(
        body,
        grid=(N_PAD // ROW_BLK,),
        in_specs=[
            pl.BlockSpec((ROW_BLK, D_HID), lambda i: (i, 0)),
            pl.BlockSpec((NC, ROW_BLK, DEG_W), lambda i: (0, i, 0)),
        ],
        out_specs=[
            pl.BlockSpec((ROW_BLK, D_HID), lambda i: (i, 0)),
            pl.BlockSpec((ROW_BLK, 1), lambda i: (i, 0)),
        ],
        out_shape=[
            jax.ShapeDtypeStruct((N_PAD, D_HID), jnp.float32),
            jax.ShapeDtypeStruct((N_PAD, 1), jnp.float32),
        ],
    )(h0, deg_parts)


def _tc_mid(acc1, h0p, dis, b0):
    def body(acc_ref, h0_ref, dis_ref, b0_ref, out_ref):
        agg = acc_ref[0] + acc_ref[1] + h0_ref[...]
        dis = dis_ref[...]
        h1 = jnp.maximum(dis * agg + b0_ref[...], 0.0)
        out_ref[...] = h1 * dis

    return pl.pallas_call(
        body,
        grid=(N_PAD // ROW_BLK,),
        in_specs=[
            pl.BlockSpec((NC, ROW_BLK, D_HID), lambda i: (0, i, 0)),
            pl.BlockSpec((ROW_BLK, D_HID), lambda i: (i, 0)),
            pl.BlockSpec((ROW_BLK, 1), lambda i: (i, 0)),
            pl.BlockSpec((1, D_HID), lambda i: (0, 0)),
        ],
        out_specs=pl.BlockSpec((ROW_BLK, D_HID), lambda i: (i, 0)),
        out_shape=jax.ShapeDtypeStruct((N_PAD, D_HID), jnp.float32),
    )(acc1, h0p, dis, b0)


def _tc_post(acc2, g1, dis, W1, b1):
    def body(acc_ref, g_ref, dis_ref, w_ref, b1_ref, out_ref):
        agg2 = dis_ref[...] * (acc_ref[0] + acc_ref[1] + g_ref[...])
        out_ref[...] = jnp.dot(agg2, w_ref[...],
                               preferred_element_type=jnp.float32) \
            + b1_ref[...]

    return pl.pallas_call(
        body,
        grid=(N_NODES // OUT_BLK,),
        in_specs=[
            pl.BlockSpec((NC, OUT_BLK, D_HID), lambda i: (0, i, 0)),
            pl.BlockSpec((OUT_BLK, D_HID), lambda i: (i, 0)),
            pl.BlockSpec((OUT_BLK, 1), lambda i: (i, 0)),
            pl.BlockSpec((D_HID, D_OUT), lambda i: (0, 0)),
            pl.BlockSpec((1, D_OUT), lambda i: (0, 0)),
        ],
        out_specs=pl.BlockSpec((OUT_BLK, D_OUT), lambda i: (i, 0)),
        out_shape=jax.ShapeDtypeStruct((N_NODES, D_OUT), jnp.float32),
    )(acc2, g1, dis, W1, b1)


def kernel(x, edge_index, W0, b0, W1, b1):
    pad_e = E_PAD - edge_index.shape[1]
    pad_idx = DUMMY + jnp.arange(pad_e, dtype=jnp.int32) % (N_PAD - N_NODES)
    pad2 = jnp.broadcast_to(pad_idx, (2, pad_e))
    ei3 = jnp.concatenate([edge_index, pad2], axis=1).reshape(
        2, NC * NS, N_CHUNKS, CHUNK)
    x_pad = jnp.pad(x, ((0, N_PAD - N_NODES), (0, 0)))

    ones_hbm = jnp.ones((CHUNK, DEG_W), dtype=jnp.float32)
    zeros_feat = jnp.zeros((N_PAD, D_HID), dtype=jnp.float32)

    deg_parts = _sc_degree(ei3, ones_hbm, zeros_feat)
    h0 = _tc_mm0(x_pad, W0)
    h0p, dis = _tc_scale(h0, deg_parts)
    acc1 = _sc_edge_pass(h0p, ei3, zeros_feat)
    g1 = _tc_mid(acc1, h0p, dis, b0.reshape(1, D_HID))
    acc2 = _sc_edge_pass(g1, ei3, zeros_feat)
    return _tc_post(acc2, g1, dis, W1, b1.reshape(1, D_OUT))


# speedup vs baseline: 1.2924x; 1.2924x over previous
import dataclasses
import functools

import jax
import jax.numpy as jnp
from jax import lax
from jax.experimental import pallas as pl
from jax.experimental.pallas import tpu as pltpu
from jax.experimental.pallas import tpu_sc as plsc

N_NODES = 10000
D_IN = 256
D_HID = 128
D_OUT = 256

NC = 2
NS = 16
CHUNK = 128

N_PAD = 10240
E_PAD = 163840
DUMMY = N_NODES

ROW_BLK = 1024
OUT_BLK = 1000
N_CHUNKS = E_PAD // (NC * NS) // CHUNK


def _vector_mesh():
    return plsc.VectorSubcoreMesh(core_axis_name="c", subcore_axis_name="s")


_SC_CP = pltpu.CompilerParams()
if "needs_layout_passes" in pltpu.CompilerParams.__dataclass_fields__:
    _SC_CP = dataclasses.replace(_SC_CP, needs_layout_passes=False)


def _sc_degree(ei3):
    NPT = N_PAD // NS

    @functools.partial(
        pl.kernel,
        mesh=_vector_mesh(),
        compiler_params=_SC_CP,
        out_type=jax.ShapeDtypeStruct((NC, N_PAD), jnp.float32),
        scratch_types=[
            pltpu.VMEM((N_CHUNKS, CHUNK), jnp.int32),
            pltpu.VMEM((N_PAD,), jnp.float32),
            pltpu.VMEM((NS, NPT), jnp.float32),
            pltpu.VMEM((NPT,), jnp.float32),
            pltpu.VMEM_SHARED((NS, N_PAD), jnp.float32),
        ],
    )
    def k(ei_h, out_h, dst_v, hist, red_v, sum_v, stage_sh):
        cid = lax.axis_index("c")
        sid = lax.axis_index("s")
        wid = cid * NS + sid
        pltpu.sync_copy(ei_h.at[1, wid], dst_v)

        zeros16 = jnp.zeros((16,), jnp.float32)

        @pl.loop(0, N_PAD // 128)
        def _(i):
            for t in range(8):
                hist[pl.ds(i * 128 + t * 16, 16)] = zeros16

        ones16 = jnp.ones((16,), jnp.float32)

        @pl.loop(0, N_CHUNKS)
        def _(j):
            for t in range(CHUNK // 16):
                idx = dst_v[j, pl.ds(t * 16, 16)]
                plsc.addupdate_scatter(hist, [idx], ones16)

        pltpu.sync_copy(hist, stage_sh.at[sid])
        plsc.subcore_barrier()
        pltpu.sync_copy(stage_sh.at[:, pl.ds(sid * NPT, NPT)], red_v)

        @pl.loop(0, NPT // 16)
        def _(g):
            acc = zeros16
            for s in range(NS):
                acc = acc + red_v[s, pl.ds(g * 16, 16)]
            sum_v[pl.ds(g * 16, 16)] = acc

        pltpu.sync_copy(sum_v, out_h.at[cid, pl.ds(sid * NPT, NPT)])

    return k(ei3)


def _sc_edge_pass(table, ei3, zeros_hbm):
    @functools.partial(
        pl.kernel,
        mesh=_vector_mesh(),
        out_type=jax.ShapeDtypeStruct((NC, N_PAD, D_HID), jnp.float32),
        scratch_types=[
            pltpu.VMEM((N_CHUNKS, CHUNK), jnp.int32),
            pltpu.VMEM((N_CHUNKS, CHUNK), jnp.int32),
            pltpu.VMEM((CHUNK, D_HID), jnp.float32),
            pltpu.VMEM((CHUNK, D_HID), jnp.float32),
            pltpu.VMEM_SHARED((N_PAD, D_HID), jnp.float32),
            pltpu.SemaphoreType.DMA,
            pltpu.SemaphoreType.DMA,
        ],
    )
    def k(tab_h, ei_h, zeros_h, out_h, src_v, dst_v, rows0, rows1,
          acc_sh, sem0, sem1):
        cid = lax.axis_index("c")
        sid = lax.axis_index("s")
        wid = cid * NS + sid
        rpt = N_PAD // NS
        pltpu.sync_copy(zeros_h.at[pl.ds(sid * rpt, rpt)],
                        acc_sh.at[pl.ds(sid * rpt, rpt)])
        pltpu.sync_copy(ei_h.at[0, wid], src_v)
        pltpu.sync_copy(ei_h.at[1, wid], dst_v)
        plsc.subcore_barrier()

        pltpu.async_copy(tab_h.at[src_v.at[0]], rows0, sem0)

        @pl.loop(0, N_CHUNKS - 2, step=2)
        def _(j):
            pltpu.async_copy(tab_h.at[src_v.at[j + 1]], rows1, sem1)
            pltpu.make_async_copy(tab_h.at[src_v.at[j]], rows0, sem0).wait()
            pltpu.sync_copy(rows0, acc_sh.at[dst_v.at[j]], add=True)
            pltpu.async_copy(tab_h.at[src_v.at[j + 2]], rows0, sem0)
            pltpu.make_async_copy(tab_h.at[src_v.at[j + 1]], rows1,
                                  sem1).wait()
            pltpu.sync_copy(rows1, acc_sh.at[dst_v.at[j + 1]], add=True)

        pltpu.async_copy(tab_h.at[src_v.at[N_CHUNKS - 1]], rows1, sem1)
        pltpu.make_async_copy(tab_h.at[src_v.at[N_CHUNKS - 2]], rows0,
                              sem0).wait()
        pltpu.sync_copy(rows0, acc_sh.at[dst_v.at[N_CHUNKS - 2]], add=True)
        pltpu.make_async_copy(tab_h.at[src_v.at[N_CHUNKS - 1]], rows1,
                              sem1).wait()
        pltpu.sync_copy(rows1, acc_sh.at[dst_v.at[N_CHUNKS - 1]], add=True)

        plsc.subcore_barrier()
        pltpu.sync_copy(acc_sh.at[pl.ds(sid * rpt, rpt)],
                        out_h.at[cid, pl.ds(sid * rpt, rpt)])

    return k(table, ei3, zeros_hbm)


def _tc_mm0(x_pad, W0):
    def body(x_ref, w_ref, h_ref):
        h_ref[...] = jnp.dot(x_ref[...], w_ref[...],
                             preferred_element_type=jnp.float32)

    return pl.pallas_call(
        body,
        grid=(N_PAD // ROW_BLK,),
        in_specs=[
            pl.BlockSpec((ROW_BLK, D_IN), lambda i: (i, 0)),
            pl.BlockSpec((D_IN, D_HID), lambda i: (0, 0)),
        ],
        out_specs=pl.BlockSpec((ROW_BLK, D_HID), lambda i: (i, 0)),
        out_shape=jax.ShapeDtypeStruct((N_PAD, D_HID), jnp.float32),
    )(x_pad, W0)


def _tc_scale(h0, deg3):
    def body(h0_ref, dg_ref, h_ref, dis_ref):
        cnt = dg_ref[0] + dg_ref[1]
        dis = lax.rsqrt(cnt + 1.0)
        h_ref[...] = h0_ref[...] * dis
        dis_ref[...] = dis

    return pl.pallas_call(
        body,
        grid=(N_PAD // ROW_BLK,),
        in_specs=[
            pl.BlockSpec((ROW_BLK, D_HID), lambda i: (i, 0)),
            pl.BlockSpec((NC, ROW_BLK, 1), lambda i: (0, i, 0)),
        ],
        out_specs=[
            pl.BlockSpec((ROW_BLK, D_HID), lambda i: (i, 0)),
            pl.BlockSpec((ROW_BLK, 1), lambda i: (i, 0)),
        ],
        out_shape=[
            jax.ShapeDtypeStruct((N_PAD, D_HID), jnp.float32),
            jax.ShapeDtypeStruct((N_PAD, 1), jnp.float32),
        ],
    )(h0, deg3)


def _tc_mid(acc1, h0p, dis, b0):
    def body(acc_ref, h0_ref, dis_ref, b0_ref, out_ref):
        agg = acc_ref[0] + acc_ref[1] + h0_ref[...]
        dis = dis_ref[...]
        h1 = jnp.maximum(dis * agg + b0_ref[...], 0.0)
        out_ref[...] = h1 * dis

    return pl.pallas_call(
        body,
        grid=(N_PAD // ROW_BLK,),
        in_specs=[
            pl.BlockSpec((NC, ROW_BLK, D_HID), lambda i: (0, i, 0)),
            pl.BlockSpec((ROW_BLK, D_HID), lambda i: (i, 0)),
            pl.BlockSpec((ROW_BLK, 1), lambda i: (i, 0)),
            pl.BlockSpec((1, D_HID), lambda i: (0, 0)),
        ],
        out_specs=pl.BlockSpec((ROW_BLK, D_HID), lambda i: (i, 0)),
        out_shape=jax.ShapeDtypeStruct((N_PAD, D_HID), jnp.float32),
    )(acc1, h0p, dis, b0)


def _tc_post(acc2, g1, dis, W1, b1):
    def body(acc_ref, g_ref, dis_ref, w_ref, b1_ref, out_ref):
        agg2 = dis_ref[...] * (acc_ref[0] + acc_ref[1] + g_ref[...])
        out_ref[...] = jnp.dot(agg2, w_ref[...],
                               preferred_element_type=jnp.float32) \
            + b1_ref[...]

    return pl.pallas_call(
        body,
        grid=(N_NODES // OUT_BLK,),
        in_specs=[
            pl.BlockSpec((NC, OUT_BLK, D_HID), lambda i: (0, i, 0)),
            pl.BlockSpec((OUT_BLK, D_HID), lambda i: (i, 0)),
            pl.BlockSpec((OUT_BLK, 1), lambda i: (i, 0)),
            pl.BlockSpec((D_HID, D_OUT), lambda i: (0, 0)),
            pl.BlockSpec((1, D_OUT), lambda i: (0, 0)),
        ],
        out_specs=pl.BlockSpec((OUT_BLK, D_OUT), lambda i: (i, 0)),
        out_shape=jax.ShapeDtypeStruct((N_NODES, D_OUT), jnp.float32),
    )(acc2, g1, dis, W1, b1)


def kernel(x, edge_index, W0, b0, W1, b1):
    pad_e = E_PAD - edge_index.shape[1]
    pad_idx = DUMMY + jnp.arange(pad_e, dtype=jnp.int32) % (N_PAD - N_NODES)
    pad2 = jnp.broadcast_to(pad_idx, (2, pad_e))
    ei3 = jnp.concatenate([edge_index, pad2], axis=1).reshape(
        2, NC * NS, N_CHUNKS, CHUNK)
    x_pad = jnp.pad(x, ((0, N_PAD - N_NODES), (0, 0)))

    zeros_feat = jnp.zeros((N_PAD, D_HID), dtype=jnp.float32)

    deg = _sc_degree(ei3)
    deg3 = deg.reshape(NC, N_PAD, 1)
    h0 = _tc_mm0(x_pad, W0)
    h0p, dis = _tc_scale(h0, deg3)
    acc1 = _sc_edge_pass(h0p, ei3, zeros_feat)
    g1 = _tc_mid(acc1, h0p, dis, b0.reshape(1, D_HID))
    acc2 = _sc_edge_pass(g1, ei3, zeros_feat)
    return _tc_post(acc2, g1, dis, W1, b1.reshape(1, D_OUT))


# speedup vs baseline: 1.3975x; 1.0813x over previous
import dataclasses
import functools

import jax
import jax.numpy as jnp
import numpy as np
from jax import lax
from jax.experimental import pallas as pl
from jax.experimental.pallas import tpu as pltpu
from jax.experimental.pallas import tpu_sc as plsc

N_NODES = 10000
D_IN = 256
D_HID = 128
D_OUT = 256

NC = 2
NS = 16
CHUNK = 128

N_PAD = 10240
E_PAD = 163840
DUMMY = N_NODES

ROW_BLK = 1024
OUT_BLK = 1000
N_CHUNKS = E_PAD // (NC * NS) // CHUNK


def _vector_mesh():
    return plsc.VectorSubcoreMesh(core_axis_name="c", subcore_axis_name="s")


_SC_CP = pltpu.CompilerParams()
if "needs_layout_passes" in pltpu.CompilerParams.__dataclass_fields__:
    _SC_CP = dataclasses.replace(_SC_CP, needs_layout_passes=False)


def _sc_degree(ei3):
    NPT = N_PAD // NS

    @functools.partial(
        pl.kernel,
        mesh=_vector_mesh(),
        compiler_params=_SC_CP,
        out_type=jax.ShapeDtypeStruct((NC, N_PAD), jnp.float32),
        scratch_types=[
            pltpu.VMEM((N_CHUNKS, CHUNK), jnp.int32),
            pltpu.VMEM((N_PAD,), jnp.float32),
            pltpu.VMEM((NS, NPT), jnp.float32),
            pltpu.VMEM((NPT,), jnp.float32),
            pltpu.VMEM_SHARED((NS, N_PAD), jnp.float32),
        ],
    )
    def k(ei_h, out_h, dst_v, hist, red_v, sum_v, stage_sh):
        cid = lax.axis_index("c")
        sid = lax.axis_index("s")
        wid = cid * NS + sid
        pltpu.sync_copy(ei_h.at[1, wid], dst_v)

        zeros16 = jnp.zeros((16,), jnp.float32)

        @pl.loop(0, N_PAD // 128)
        def _(i):
            for t in range(8):
                hist[pl.ds(i * 128 + t * 16, 16)] = zeros16

        ones16 = jnp.ones((16,), jnp.float32)

        @pl.loop(0, N_CHUNKS)
        def _(j):
            for t in range(CHUNK // 16):
                idx = dst_v[j, pl.ds(t * 16, 16)]
                plsc.addupdate_scatter(hist, [idx], ones16)

        pltpu.sync_copy(hist, stage_sh.at[sid])
        plsc.subcore_barrier()
        pltpu.sync_copy(stage_sh.at[:, pl.ds(sid * NPT, NPT)], red_v)

        @pl.loop(0, NPT // 16)
        def _(g):
            acc = zeros16
            for s in range(NS):
                acc = acc + red_v[s, pl.ds(g * 16, 16)]
            sum_v[pl.ds(g * 16, 16)] = acc

        pltpu.sync_copy(sum_v, out_h.at[cid, pl.ds(sid * NPT, NPT)])

    return k(ei3)


def _sc_edge_pass(table, ei3, zeros_hbm):
    @functools.partial(
        pl.kernel,
        mesh=_vector_mesh(),
        out_type=jax.ShapeDtypeStruct((NC, N_PAD, D_HID), jnp.float32),
        scratch_types=[
            pltpu.VMEM((N_CHUNKS, CHUNK), jnp.int32),
            pltpu.VMEM((N_CHUNKS, CHUNK), jnp.int32),
            pltpu.VMEM((CHUNK, D_HID), jnp.float32),
            pltpu.VMEM((CHUNK, D_HID), jnp.float32),
            pltpu.VMEM_SHARED((N_PAD, D_HID), jnp.float32),
            pltpu.SemaphoreType.DMA,
            pltpu.SemaphoreType.DMA,
        ],
    )
    def k(tab_h, ei_h, zeros_h, out_h, src_v, dst_v, rows0, rows1,
          acc_sh, sem0, sem1):
        cid = lax.axis_index("c")
        sid = lax.axis_index("s")
        wid = cid * NS + sid
        rpt = N_PAD // NS
        pltpu.sync_copy(zeros_h.at[pl.ds(sid * rpt, rpt)],
                        acc_sh.at[pl.ds(sid * rpt, rpt)])
        pltpu.sync_copy(ei_h.at[0, wid], src_v)
        pltpu.sync_copy(ei_h.at[1, wid], dst_v)
        plsc.subcore_barrier()

        pltpu.async_copy(tab_h.at[src_v.at[0]], rows0, sem0)

        @pl.loop(0, N_CHUNKS - 2, step=2)
        def _(j):
            pltpu.async_copy(tab_h.at[src_v.at[j + 1]], rows1, sem1)
            pltpu.make_async_copy(tab_h.at[src_v.at[j]], rows0, sem0).wait()
            pltpu.sync_copy(rows0, acc_sh.at[dst_v.at[j]], add=True)
            pltpu.async_copy(tab_h.at[src_v.at[j + 2]], rows0, sem0)
            pltpu.make_async_copy(tab_h.at[src_v.at[j + 1]], rows1,
                                  sem1).wait()
            pltpu.sync_copy(rows1, acc_sh.at[dst_v.at[j + 1]], add=True)

        pltpu.async_copy(tab_h.at[src_v.at[N_CHUNKS - 1]], rows1, sem1)
        pltpu.make_async_copy(tab_h.at[src_v.at[N_CHUNKS - 2]], rows0,
                              sem0).wait()
        pltpu.sync_copy(rows0, acc_sh.at[dst_v.at[N_CHUNKS - 2]], add=True)
        pltpu.make_async_copy(tab_h.at[src_v.at[N_CHUNKS - 1]], rows1,
                              sem1).wait()
        pltpu.sync_copy(rows1, acc_sh.at[dst_v.at[N_CHUNKS - 1]], add=True)

        plsc.subcore_barrier()
        pltpu.sync_copy(acc_sh.at[pl.ds(sid * rpt, rpt)],
                        out_h.at[cid, pl.ds(sid * rpt, rpt)])

    return k(table, ei3, zeros_hbm)


def _tc_mm0_scale(x_pad, W0, deg3):
    def body(x_ref, w_ref, dg_ref, h_ref, dis_ref):
        cntT = jnp.transpose(dg_ref[...])
        cnt = cntT[:, 0:1] + cntT[:, 1:2]
        dis = lax.rsqrt(cnt + 1.0)
        h = jnp.dot(x_ref[...], w_ref[...],
                    preferred_element_type=jnp.float32)
        h_ref[...] = h * dis
        dis_ref[...] = dis

    return pl.pallas_call(
        body,
        grid=(N_PAD // ROW_BLK,),
        in_specs=[
            pl.BlockSpec((ROW_BLK, D_IN), lambda i: (i, 0)),
            pl.BlockSpec((D_IN, D_HID), lambda i: (0, 0)),
            pl.BlockSpec((NC, ROW_BLK), lambda i: (0, i)),
        ],
        out_specs=[
            pl.BlockSpec((ROW_BLK, D_HID), lambda i: (i, 0)),
            pl.BlockSpec((ROW_BLK, 1), lambda i: (i, 0)),
        ],
        out_shape=[
            jax.ShapeDtypeStruct((N_PAD, D_HID), jnp.float32),
            jax.ShapeDtypeStruct((N_PAD, 1), jnp.float32),
        ],
    )(x_pad, W0, deg3)


def _tc_mid(acc1, h0p, dis, b0):
    def body(acc_ref, h0_ref, dis_ref, b0_ref, out_ref):
        agg = acc_ref[0] + acc_ref[1] + h0_ref[...]
        dis = dis_ref[...]
        h1 = jnp.maximum(dis * agg + b0_ref[...], 0.0)
        out_ref[...] = h1 * dis

    return pl.pallas_call(
        body,
        grid=(N_PAD // ROW_BLK,),
        in_specs=[
            pl.BlockSpec((NC, ROW_BLK, D_HID), lambda i: (0, i, 0)),
            pl.BlockSpec((ROW_BLK, D_HID), lambda i: (i, 0)),
            pl.BlockSpec((ROW_BLK, 1), lambda i: (i, 0)),
            pl.BlockSpec((1, D_HID), lambda i: (0, 0)),
        ],
        out_specs=pl.BlockSpec((ROW_BLK, D_HID), lambda i: (i, 0)),
        out_shape=jax.ShapeDtypeStruct((N_PAD, D_HID), jnp.float32),
    )(acc1, h0p, dis, b0)


def _tc_post(acc2, g1, dis, W1, b1):
    def body(acc_ref, g_ref, dis_ref, w_ref, b1_ref, out_ref):
        agg2 = dis_ref[...] * (acc_ref[0] + acc_ref[1] + g_ref[...])
        out_ref[...] = jnp.dot(agg2, w_ref[...],
                               preferred_element_type=jnp.float32) \
            + b1_ref[...]

    return pl.pallas_call(
        body,
        grid=(N_NODES // OUT_BLK,),
        in_specs=[
            pl.BlockSpec((NC, OUT_BLK, D_HID), lambda i: (0, i, 0)),
            pl.BlockSpec((OUT_BLK, D_HID), lambda i: (i, 0)),
            pl.BlockSpec((OUT_BLK, 1), lambda i: (i, 0)),
            pl.BlockSpec((D_HID, D_OUT), lambda i: (0, 0)),
            pl.BlockSpec((1, D_OUT), lambda i: (0, 0)),
        ],
        out_specs=pl.BlockSpec((OUT_BLK, D_OUT), lambda i: (i, 0)),
        out_shape=jax.ShapeDtypeStruct((N_NODES, D_OUT), jnp.float32),
    )(acc2, g1, dis, W1, b1)


_ZEROS_FEAT = np.zeros((N_PAD, D_HID), dtype=np.float32)


def kernel(x, edge_index, W0, b0, W1, b1):
    pad_e = E_PAD - edge_index.shape[1]
    pad_idx = DUMMY + jnp.arange(pad_e, dtype=jnp.int32) % (N_PAD - N_NODES)
    pad2 = jnp.broadcast_to(pad_idx, (2, pad_e))
    ei3 = jnp.concatenate([edge_index, pad2], axis=1).reshape(
        2, NC * NS, N_CHUNKS, CHUNK)
    x_pad = jnp.pad(x, ((0, N_PAD - N_NODES), (0, 0)))

    zeros_feat = _ZEROS_FEAT

    deg3 = _sc_degree(ei3)
    h0p, dis = _tc_mm0_scale(x_pad, W0, deg3)
    acc1 = _sc_edge_pass(h0p, ei3, zeros_feat)
    g1 = _tc_mid(acc1, h0p, dis, b0.reshape(1, D_HID))
    acc2 = _sc_edge_pass(g1, ei3, zeros_feat)
    return _tc_post(acc2, g1, dis, W1, b1.reshape(1, D_OUT))


# speedup vs baseline: 1.4196x; 1.0158x over previous
import dataclasses
import functools

import jax
import jax.numpy as jnp
import numpy as np
from jax import lax
from jax.experimental import pallas as pl
from jax.experimental.pallas import tpu as pltpu
from jax.experimental.pallas import tpu_sc as plsc

N_NODES = 10000
D_IN = 256
D_HID = 128
D_OUT = 256

NC = 2
NS = 16
CHUNK = 128

N_PAD = 10240
E_PAD = 163840
DUMMY = N_NODES

ROW_BLK = 2048
OUT_BLK = 1000
N_CHUNKS = E_PAD // (NC * NS) // CHUNK


def _vector_mesh():
    return plsc.VectorSubcoreMesh(core_axis_name="c", subcore_axis_name="s")


_SC_CP = pltpu.CompilerParams()
if "needs_layout_passes" in pltpu.CompilerParams.__dataclass_fields__:
    _SC_CP = dataclasses.replace(_SC_CP, needs_layout_passes=False)


def _sc_degree(ei3):
    NPT = N_PAD // NS

    @functools.partial(
        pl.kernel,
        mesh=_vector_mesh(),
        compiler_params=_SC_CP,
        out_type=jax.ShapeDtypeStruct((NC, N_PAD), jnp.float32),
        scratch_types=[
            pltpu.VMEM((N_CHUNKS, CHUNK), jnp.int32),
            pltpu.VMEM((N_PAD,), jnp.float32),
            pltpu.VMEM((NS, NPT), jnp.float32),
            pltpu.VMEM((NPT,), jnp.float32),
            pltpu.VMEM_SHARED((NS, N_PAD), jnp.float32),
        ],
    )
    def k(ei_h, out_h, dst_v, hist, red_v, sum_v, stage_sh):
        cid = lax.axis_index("c")
        sid = lax.axis_index("s")
        wid = cid * NS + sid
        pltpu.sync_copy(ei_h.at[1, wid], dst_v)

        zeros16 = jnp.zeros((16,), jnp.float32)

        @pl.loop(0, N_PAD // 128)
        def _(i):
            for t in range(8):
                hist[pl.ds(i * 128 + t * 16, 16)] = zeros16

        ones16 = jnp.ones((16,), jnp.float32)

        @pl.loop(0, N_CHUNKS)
        def _(j):
            for t in range(CHUNK // 16):
                idx = dst_v[j, pl.ds(t * 16, 16)]
                plsc.addupdate_scatter(hist, [idx], ones16)

        pltpu.sync_copy(hist, stage_sh.at[sid])
        plsc.subcore_barrier()
        pltpu.sync_copy(stage_sh.at[:, pl.ds(sid * NPT, NPT)], red_v)

        @pl.loop(0, NPT // 16)
        def _(g):
            acc = zeros16
            for s in range(NS):
                acc = acc + red_v[s, pl.ds(g * 16, 16)]
            sum_v[pl.ds(g * 16, 16)] = acc

        pltpu.sync_copy(sum_v, out_h.at[cid, pl.ds(sid * NPT, NPT)])

    return k(ei3)


def _sc_edge_pass(table, ei3, zeros_hbm):
    @functools.partial(
        pl.kernel,
        mesh=_vector_mesh(),
        out_type=jax.ShapeDtypeStruct((NC, N_PAD, D_HID), jnp.float32),
        scratch_types=[
            pltpu.VMEM((N_CHUNKS, CHUNK), jnp.int32),
            pltpu.VMEM((N_CHUNKS, CHUNK), jnp.int32),
            pltpu.VMEM((CHUNK, D_HID), jnp.float32),
            pltpu.VMEM((CHUNK, D_HID), jnp.float32),
            pltpu.VMEM_SHARED((N_PAD, D_HID), jnp.float32),
            pltpu.SemaphoreType.DMA,
            pltpu.SemaphoreType.DMA,
        ],
    )
    def k(tab_h, ei_h, zeros_h, out_h, src_v, dst_v, rows0, rows1,
          acc_sh, sem0, sem1):
        cid = lax.axis_index("c")
        sid = lax.axis_index("s")
        wid = cid * NS + sid
        rpt = N_PAD // NS
        pltpu.sync_copy(zeros_h.at[pl.ds(sid * rpt, rpt)],
                        acc_sh.at[pl.ds(sid * rpt, rpt)])
        pltpu.sync_copy(ei_h.at[0, wid], src_v)
        pltpu.sync_copy(ei_h.at[1, wid], dst_v)
        plsc.subcore_barrier()

        pltpu.async_copy(tab_h.at[src_v.at[0]], rows0, sem0)

        @pl.loop(0, N_CHUNKS - 2, step=2)
        def _(j):
            pltpu.async_copy(tab_h.at[src_v.at[j + 1]], rows1, sem1)
            pltpu.make_async_copy(tab_h.at[src_v.at[j]], rows0, sem0).wait()
            pltpu.sync_copy(rows0, acc_sh.at[dst_v.at[j]], add=True)
            pltpu.async_copy(tab_h.at[src_v.at[j + 2]], rows0, sem0)
            pltpu.make_async_copy(tab_h.at[src_v.at[j + 1]], rows1,
                                  sem1).wait()
            pltpu.sync_copy(rows1, acc_sh.at[dst_v.at[j + 1]], add=True)

        pltpu.async_copy(tab_h.at[src_v.at[N_CHUNKS - 1]], rows1, sem1)
        pltpu.make_async_copy(tab_h.at[src_v.at[N_CHUNKS - 2]], rows0,
                              sem0).wait()
        pltpu.sync_copy(rows0, acc_sh.at[dst_v.at[N_CHUNKS - 2]], add=True)
        pltpu.make_async_copy(tab_h.at[src_v.at[N_CHUNKS - 1]], rows1,
                              sem1).wait()
        pltpu.sync_copy(rows1, acc_sh.at[dst_v.at[N_CHUNKS - 1]], add=True)

        plsc.subcore_barrier()
        pltpu.sync_copy(acc_sh.at[pl.ds(sid * rpt, rpt)],
                        out_h.at[cid, pl.ds(sid * rpt, rpt)])

    return k(table, ei3, zeros_hbm)


def _tc_mm0_scale(x_pad, W0, deg3):
    def body(x_ref, w_ref, dg_ref, h_ref, dis_ref):
        cntT = jnp.transpose(dg_ref[...])
        cnt = cntT[:, 0:1] + cntT[:, 1:2]
        dis = lax.rsqrt(cnt + 1.0)
        h = jnp.dot(x_ref[...], w_ref[...],
                    preferred_element_type=jnp.float32)
        h_ref[...] = h * dis
        dis_ref[...] = dis

    return pl.pallas_call(
        body,
        grid=(N_PAD // ROW_BLK,),
        in_specs=[
            pl.BlockSpec((ROW_BLK, D_IN), lambda i: (i, 0)),
            pl.BlockSpec((D_IN, D_HID), lambda i: (0, 0)),
            pl.BlockSpec((NC, ROW_BLK), lambda i: (0, i)),
        ],
        out_specs=[
            pl.BlockSpec((ROW_BLK, D_HID), lambda i: (i, 0)),
            pl.BlockSpec((ROW_BLK, 1), lambda i: (i, 0)),
        ],
        out_shape=[
            jax.ShapeDtypeStruct((N_PAD, D_HID), jnp.float32),
            jax.ShapeDtypeStruct((N_PAD, 1), jnp.float32),
        ],
    )(x_pad, W0, deg3)


def _tc_mid(acc1, h0p, dis, b0):
    def body(acc_ref, h0_ref, dis_ref, b0_ref, out_ref):
        agg = acc_ref[0] + acc_ref[1] + h0_ref[...]
        dis = dis_ref[...]
        h1 = jnp.maximum(dis * agg + b0_ref[...], 0.0)
        out_ref[...] = h1 * dis

    return pl.pallas_call(
        body,
        grid=(N_PAD // ROW_BLK,),
        in_specs=[
            pl.BlockSpec((NC, ROW_BLK, D_HID), lambda i: (0, i, 0)),
            pl.BlockSpec((ROW_BLK, D_HID), lambda i: (i, 0)),
            pl.BlockSpec((ROW_BLK, 1), lambda i: (i, 0)),
            pl.BlockSpec((1, D_HID), lambda i: (0, 0)),
        ],
        out_specs=pl.BlockSpec((ROW_BLK, D_HID), lambda i: (i, 0)),
        out_shape=jax.ShapeDtypeStruct((N_PAD, D_HID), jnp.float32),
    )(acc1, h0p, dis, b0)


def _tc_post(acc2, g1, dis, W1, b1):
    def body(acc_ref, g_ref, dis_ref, w_ref, b1_ref, out_ref):
        agg2 = dis_ref[...] * (acc_ref[0] + acc_ref[1] + g_ref[...])
        out_ref[...] = jnp.dot(agg2, w_ref[...],
                               preferred_element_type=jnp.float32) \
            + b1_ref[...]

    return pl.pallas_call(
        body,
        grid=(N_NODES // OUT_BLK,),
        in_specs=[
            pl.BlockSpec((NC, OUT_BLK, D_HID), lambda i: (0, i, 0)),
            pl.BlockSpec((OUT_BLK, D_HID), lambda i: (i, 0)),
            pl.BlockSpec((OUT_BLK, 1), lambda i: (i, 0)),
            pl.BlockSpec((D_HID, D_OUT), lambda i: (0, 0)),
            pl.BlockSpec((1, D_OUT), lambda i: (0, 0)),
        ],
        out_specs=pl.BlockSpec((OUT_BLK, D_OUT), lambda i: (i, 0)),
        out_shape=jax.ShapeDtypeStruct((N_NODES, D_OUT), jnp.float32),
    )(acc2, g1, dis, W1, b1)


_ZEROS_FEAT = np.zeros((N_PAD, D_HID), dtype=np.float32)


def kernel(x, edge_index, W0, b0, W1, b1):
    pad_e = E_PAD - edge_index.shape[1]
    pad_idx = DUMMY + jnp.arange(pad_e, dtype=jnp.int32) % (N_PAD - N_NODES)
    pad2 = jnp.broadcast_to(pad_idx, (2, pad_e))
    ei3 = jnp.concatenate([edge_index, pad2], axis=1).reshape(
        2, NC * NS, N_CHUNKS, CHUNK)
    x_pad = jnp.pad(x, ((0, N_PAD - N_NODES), (0, 0)))

    zeros_feat = _ZEROS_FEAT

    deg3 = _sc_degree(ei3)
    h0p, dis = _tc_mm0_scale(x_pad, W0, deg3)
    acc1 = _sc_edge_pass(h0p, ei3, zeros_feat)
    g1 = _tc_mid(acc1, h0p, dis, b0.reshape(1, D_HID))
    acc2 = _sc_edge_pass(g1, ei3, zeros_feat)
    return _tc_post(acc2, g1, dis, W1, b1.reshape(1, D_OUT))


# speedup vs baseline: 1.4291x; 1.0067x over previous
import dataclasses
import functools

import jax
import jax.numpy as jnp
import numpy as np
from jax import lax
from jax.experimental import pallas as pl
from jax.experimental.pallas import tpu as pltpu
from jax.experimental.pallas import tpu_sc as plsc

N_NODES = 10000
D_IN = 256
D_HID = 128
D_OUT = 256

NC = 2
NS = 16
CHUNK = 128

N_PAD = 10240
E_PAD = 163840
DUMMY = N_NODES

ROW_BLK = 2048
OUT_BLK = 2000
N_CHUNKS = E_PAD // (NC * NS) // CHUNK


def _vector_mesh():
    return plsc.VectorSubcoreMesh(core_axis_name="c", subcore_axis_name="s")


_SC_CP = pltpu.CompilerParams()
if "needs_layout_passes" in pltpu.CompilerParams.__dataclass_fields__:
    _SC_CP = dataclasses.replace(_SC_CP, needs_layout_passes=False)


def _sc_degree(ei3):
    NPT = N_PAD // NS

    @functools.partial(
        pl.kernel,
        mesh=_vector_mesh(),
        compiler_params=_SC_CP,
        out_type=jax.ShapeDtypeStruct((NC, N_PAD), jnp.float32),
        scratch_types=[
            pltpu.VMEM((N_CHUNKS, CHUNK), jnp.int32),
            pltpu.VMEM((N_PAD,), jnp.float32),
            pltpu.VMEM((NS, NPT), jnp.float32),
            pltpu.VMEM((NPT,), jnp.float32),
            pltpu.VMEM_SHARED((NS, N_PAD), jnp.float32),
        ],
    )
    def k(ei_h, out_h, dst_v, hist, red_v, sum_v, stage_sh):
        cid = lax.axis_index("c")
        sid = lax.axis_index("s")
        wid = cid * NS + sid
        pltpu.sync_copy(ei_h.at[1, wid], dst_v)

        zeros16 = jnp.zeros((16,), jnp.float32)

        @pl.loop(0, N_PAD // 128)
        def _(i):
            for t in range(8):
                hist[pl.ds(i * 128 + t * 16, 16)] = zeros16

        ones16 = jnp.ones((16,), jnp.float32)

        @pl.loop(0, N_CHUNKS)
        def _(j):
            for t in range(CHUNK // 16):
                idx = dst_v[j, pl.ds(t * 16, 16)]
                plsc.addupdate_scatter(hist, [idx], ones16)

        pltpu.sync_copy(hist, stage_sh.at[sid])
        plsc.subcore_barrier()
        pltpu.sync_copy(stage_sh.at[:, pl.ds(sid * NPT, NPT)], red_v)

        @pl.loop(0, NPT // 16)
        def _(g):
            acc = zeros16
            for s in range(NS):
                acc = acc + red_v[s, pl.ds(g * 16, 16)]
            sum_v[pl.ds(g * 16, 16)] = acc

        pltpu.sync_copy(sum_v, out_h.at[cid, pl.ds(sid * NPT, NPT)])

    return k(ei3)


def _sc_edge_pass(table, ei3, zeros_hbm):
    @functools.partial(
        pl.kernel,
        mesh=_vector_mesh(),
        out_type=jax.ShapeDtypeStruct((NC, N_PAD, D_HID), jnp.float32),
        scratch_types=[
            pltpu.VMEM((N_CHUNKS, CHUNK), jnp.int32),
            pltpu.VMEM((N_CHUNKS, CHUNK), jnp.int32),
            pltpu.VMEM((CHUNK, D_HID), jnp.float32),
            pltpu.VMEM((CHUNK, D_HID), jnp.float32),
            pltpu.VMEM_SHARED((N_PAD, D_HID), jnp.float32),
            pltpu.SemaphoreType.DMA,
            pltpu.SemaphoreType.DMA,
        ],
    )
    def k(tab_h, ei_h, zeros_h, out_h, src_v, dst_v, rows0, rows1,
          acc_sh, sem0, sem1):
        cid = lax.axis_index("c")
        sid = lax.axis_index("s")
        wid = cid * NS + sid
        rpt = N_PAD // NS
        pltpu.sync_copy(zeros_h.at[pl.ds(sid * rpt, rpt)],
                        acc_sh.at[pl.ds(sid * rpt, rpt)])
        pltpu.sync_copy(ei_h.at[0, wid], src_v)
        pltpu.sync_copy(ei_h.at[1, wid], dst_v)
        plsc.subcore_barrier()

        pltpu.async_copy(tab_h.at[src_v.at[0]], rows0, sem0)

        @pl.loop(0, N_CHUNKS - 2, step=2)
        def _(j):
            pltpu.async_copy(tab_h.at[src_v.at[j + 1]], rows1, sem1)
            pltpu.make_async_copy(tab_h.at[src_v.at[j]], rows0, sem0).wait()
            pltpu.sync_copy(rows0, acc_sh.at[dst_v.at[j]], add=True)
            pltpu.async_copy(tab_h.at[src_v.at[j + 2]], rows0, sem0)
            pltpu.make_async_copy(tab_h.at[src_v.at[j + 1]], rows1,
                                  sem1).wait()
            pltpu.sync_copy(rows1, acc_sh.at[dst_v.at[j + 1]], add=True)

        pltpu.async_copy(tab_h.at[src_v.at[N_CHUNKS - 1]], rows1, sem1)
        pltpu.make_async_copy(tab_h.at[src_v.at[N_CHUNKS - 2]], rows0,
                              sem0).wait()
        pltpu.sync_copy(rows0, acc_sh.at[dst_v.at[N_CHUNKS - 2]], add=True)
        pltpu.make_async_copy(tab_h.at[src_v.at[N_CHUNKS - 1]], rows1,
                              sem1).wait()
        pltpu.sync_copy(rows1, acc_sh.at[dst_v.at[N_CHUNKS - 1]], add=True)

        plsc.subcore_barrier()
        pltpu.sync_copy(acc_sh.at[pl.ds(sid * rpt, rpt)],
                        out_h.at[cid, pl.ds(sid * rpt, rpt)])

    return k(table, ei3, zeros_hbm)


def _tc_mm0_scale(x_pad, W0, deg3):
    def body(x_ref, w_ref, dg_ref, h_ref, dis_ref):
        cntT = jnp.transpose(dg_ref[...])
        cnt = cntT[:, 0:1] + cntT[:, 1:2]
        dis = lax.rsqrt(cnt + 1.0)
        h = jnp.dot(x_ref[...], w_ref[...],
                    preferred_element_type=jnp.float32)
        h_ref[...] = h * dis
        dis_ref[...] = dis

    return pl.pallas_call(
        body,
        grid=(N_PAD // ROW_BLK,),
        in_specs=[
            pl.BlockSpec((ROW_BLK, D_IN), lambda i: (i, 0)),
            pl.BlockSpec((D_IN, D_HID), lambda i: (0, 0)),
            pl.BlockSpec((NC, ROW_BLK), lambda i: (0, i)),
        ],
        out_specs=[
            pl.BlockSpec((ROW_BLK, D_HID), lambda i: (i, 0)),
            pl.BlockSpec((ROW_BLK, 1), lambda i: (i, 0)),
        ],
        out_shape=[
            jax.ShapeDtypeStruct((N_PAD, D_HID), jnp.float32),
            jax.ShapeDtypeStruct((N_PAD, 1), jnp.float32),
        ],
    )(x_pad, W0, deg3)


def _tc_mid(acc1, h0p, dis, b0):
    def body(acc_ref, h0_ref, dis_ref, b0_ref, out_ref):
        agg = acc_ref[0] + acc_ref[1] + h0_ref[...]
        dis = dis_ref[...]
        h1 = jnp.maximum(dis * agg + b0_ref[...], 0.0)
        out_ref[...] = h1 * dis

    return pl.pallas_call(
        body,
        grid=(N_PAD // ROW_BLK,),
        in_specs=[
            pl.BlockSpec((NC, ROW_BLK, D_HID), lambda i: (0, i, 0)),
            pl.BlockSpec((ROW_BLK, D_HID), lambda i: (i, 0)),
            pl.BlockSpec((ROW_BLK, 1), lambda i: (i, 0)),
            pl.BlockSpec((1, D_HID), lambda i: (0, 0)),
        ],
        out_specs=pl.BlockSpec((ROW_BLK, D_HID), lambda i: (i, 0)),
        out_shape=jax.ShapeDtypeStruct((N_PAD, D_HID), jnp.float32),
    )(acc1, h0p, dis, b0)


def _tc_post(acc2, g1, dis, W1, b1):
    def body(acc_ref, g_ref, dis_ref, w_ref, b1_ref, out_ref):
        agg2 = dis_ref[...] * (acc_ref[0] + acc_ref[1] + g_ref[...])
        out_ref[...] = jnp.dot(agg2, w_ref[...],
                               preferred_element_type=jnp.float32) \
            + b1_ref[...]

    return pl.pallas_call(
        body,
        grid=(N_NODES // OUT_BLK,),
        in_specs=[
            pl.BlockSpec((NC, OUT_BLK, D_HID), lambda i: (0, i, 0)),
            pl.BlockSpec((OUT_BLK, D_HID), lambda i: (i, 0)),
            pl.BlockSpec((OUT_BLK, 1), lambda i: (i, 0)),
            pl.BlockSpec((D_HID, D_OUT), lambda i: (0, 0)),
            pl.BlockSpec((1, D_OUT), lambda i: (0, 0)),
        ],
        out_specs=pl.BlockSpec((OUT_BLK, D_OUT), lambda i: (i, 0)),
        out_shape=jax.ShapeDtypeStruct((N_NODES, D_OUT), jnp.float32),
    )(acc2, g1, dis, W1, b1)


_ZEROS_FEAT = np.zeros((N_PAD, D_HID), dtype=np.float32)


def kernel(x, edge_index, W0, b0, W1, b1):
    pad_e = E_PAD - edge_index.shape[1]
    pad_idx = DUMMY + jnp.arange(pad_e, dtype=jnp.int32) % (N_PAD - N_NODES)
    pad2 = jnp.broadcast_to(pad_idx, (2, pad_e))
    ei3 = jnp.concatenate([edge_index, pad2], axis=1).reshape(
        2, NC * NS, N_CHUNKS, CHUNK)
    x_pad = jnp.pad(x, ((0, N_PAD - N_NODES), (0, 0)))

    zeros_feat = _ZEROS_FEAT

    deg3 = _sc_degree(ei3)
    h0p, dis = _tc_mm0_scale(x_pad, W0, deg3)
    acc1 = _sc_edge_pass(h0p, ei3, zeros_feat)
    g1 = _tc_mid(acc1, h0p, dis, b0.reshape(1, D_HID))
    acc2 = _sc_edge_pass(g1, ei3, zeros_feat)
    return _tc_post(acc2, g1, dis, W1, b1.reshape(1, D_OUT))
